# Initial kernel scaffold; baseline (speedup 1.0000x reference)
import jax
import jax.numpy as jnp
from jax.experimental import pallas as pl


def kernel(x, W_char, W_role, W_buff):
    raise NotImplementedError("write your pallas kernel here")


# speedup vs baseline: 1.3223x; 1.3223x over previous
import functools

import jax
import jax.numpy as jnp
from jax import lax
from jax.experimental import pallas as pl
from jax.experimental.pallas import tpu as pltpu
from jax.experimental.pallas import tpu_sc as plsc

B = 16384
NC, NS, L = 2, 16, 16
NW = NC * NS
RPW = B // NW
NG = RPW // L
NSTREAM = 4

_mesh = plsc.VectorSubcoreMesh(
    core_axis_name="c", subcore_axis_name="s", num_cores=NC, num_subcores=NS
)


@functools.partial(
    pl.kernel,
    out_type=(
        jax.ShapeDtypeStruct((B, 16), jnp.float32),
        jax.ShapeDtypeStruct((B, 8), jnp.float32),
        jax.ShapeDtypeStruct((B, 6), jnp.float32),
    ),
    mesh=_mesh,
    compiler_params=pltpu.CompilerParams(
        use_tc_tiling_on_sc=False, needs_layout_passes=False
    ),
    scratch_types=[
        pltpu.VMEM((RPW, 8), jnp.float32),
        pltpu.VMEM((NSTREAM, 128), jnp.int32),
        pltpu.VMEM((RPW,), jnp.int32),
        pltpu.VMEM((RPW,), jnp.int32),
        pltpu.VMEM((RPW, 16), jnp.float32),
        pltpu.VMEM((RPW, 8), jnp.float32),
        pltpu.VMEM((RPW, 6), jnp.float32),
        pltpu.VMEM((8, 8), jnp.float32),
        pltpu.VMEM((50, 6), jnp.float32),
        pltpu.SemaphoreType.DMA,
    ],
)
def _sc_embed(x_hbm, wc_hbm, wr_hbm, wb_hbm,
              out_char, out_role, out_buff,
              xv, idxc_v, idxr_v, idxb_v, char_v, role_v, buff_v,
              wr_v, wb_v, sem):
    wid = lax.axis_index("s") * NC + lax.axis_index("c")
    base = wid * RPW

    pltpu.sync_copy(x_hbm.at[pl.ds(base, RPW), pl.ds(0, 8)], xv)
    pltpu.sync_copy(wr_hbm, wr_v)
    pltpu.sync_copy(wb_hbm, wb_v)

    iota = lax.iota(jnp.int32, L)
    for g in range(NG):
        rows = iota + g * L
        c0 = plsc.load_gather(xv, [rows, jnp.full((L,), 0, jnp.int32)])
        c1 = plsc.load_gather(xv, [rows, jnp.full((L,), 1, jnp.int32)])
        c2 = plsc.load_gather(xv, [rows, jnp.full((L,), 2, jnp.int32)])
        idxc_v[g // 8, pl.ds((g % 8) * L, L)] = c0.astype(jnp.int32)
        idxr_v[pl.ds(g * L, L)] = c1.astype(jnp.int32)
        idxb_v[pl.ds(g * L, L)] = c2.astype(jnp.int32)

    char_cps = [
        pltpu.async_copy(
            wc_hbm.at[idxc_v.at[j]], char_v.at[pl.ds(j * 128, 128)], sem
        )
        for j in range(NSTREAM)
    ]

    for g in range(NG):
        rows = iota + g * L
        ir = idxr_v[pl.ds(g * L, L)]
        ib = idxb_v[pl.ds(g * L, L)]
        for c in range(8):
            cc = jnp.full((L,), c, jnp.int32)
            plsc.store_scatter(role_v, [rows, cc],
                               plsc.load_gather(wr_v, [ir, cc]))
        for c in range(6):
            cc = jnp.full((L,), c, jnp.int32)
            plsc.store_scatter(buff_v, [rows, cc],
                               plsc.load_gather(wb_v, [ib, cc]))

    for cp in char_cps:
        cp.wait()

    pltpu.sync_copy(char_v, out_char.at[pl.ds(base, RPW)])
    pltpu.sync_copy(role_v, out_role.at[pl.ds(base, RPW)])
    pltpu.sync_copy(buff_v, out_buff.at[pl.ds(base, RPW)])


def _states_body(x_ref, o_ref):
    o_ref[...] = x_ref[:, 3:76]


_STATES_BLOCK = 2048


def _states(x):
    return pl.pallas_call(
        _states_body,
        grid=(B // _STATES_BLOCK,),
        in_specs=[pl.BlockSpec((_STATES_BLOCK, 76), lambda i: (i, 0))],
        out_specs=pl.BlockSpec((_STATES_BLOCK, 73), lambda i: (i, 0)),
        out_shape=jax.ShapeDtypeStruct((B, 73), jnp.float32),
    )(x)


def kernel(x, W_char, W_role, W_buff):
    my_char, my_role, my_buff = _sc_embed(x, W_char, W_role, W_buff)
    my_states = _states(x)
    return (my_char, my_role, my_buff, my_states)


# speedup vs baseline: 2.2377x; 1.6923x over previous
import functools

import jax
import jax.numpy as jnp
from jax import lax
from jax.experimental import pallas as pl
from jax.experimental.pallas import tpu as pltpu
from jax.experimental.pallas import tpu_sc as plsc

B = 16384
NC, NS, L = 2, 16, 16
NW = NC * NS
RPW = B // NW
NG = RPW // L
NSTREAM = 4

_mesh = plsc.VectorSubcoreMesh(
    core_axis_name="c", subcore_axis_name="s", num_cores=NC, num_subcores=NS
)


@functools.partial(
    pl.kernel,
    out_type=(
        jax.ShapeDtypeStruct((B, 16), jnp.float32),
        jax.ShapeDtypeStruct((B, 8), jnp.float32),
        jax.ShapeDtypeStruct((B, 6), jnp.float32),
    ),
    mesh=_mesh,
    compiler_params=pltpu.CompilerParams(
        use_tc_tiling_on_sc=False, needs_layout_passes=False
    ),
    scratch_types=[
        pltpu.VMEM((RPW, 76), jnp.float32),
        pltpu.VMEM((RPW, 16), jnp.float32),
        pltpu.VMEM((RPW, 8), jnp.float32),
        pltpu.VMEM((RPW, 6), jnp.float32),
        pltpu.VMEM((100, 16), jnp.float32),
        pltpu.VMEM((8, 8), jnp.float32),
        pltpu.VMEM((50, 6), jnp.float32),
    ],
)
def _sc_embed(x_hbm, wc_hbm, wr_hbm, wb_hbm,
              out_char, out_role, out_buff,
              xv, char_v, role_v, buff_v, wc_v, wr_v, wb_v):
    wid = lax.axis_index("s") * NC + lax.axis_index("c")
    base = wid * RPW

    pltpu.sync_copy(x_hbm.at[pl.ds(base, RPW)], xv)
    pltpu.sync_copy(wc_hbm, wc_v)
    pltpu.sync_copy(wr_hbm, wr_v)
    pltpu.sync_copy(wb_hbm, wb_v)

    iota = lax.iota(jnp.int32, L)
    for g in range(NG):
        rows = iota + g * L
        ic = plsc.load_gather(
            xv, [rows, jnp.full((L,), 0, jnp.int32)]).astype(jnp.int32)
        ir = plsc.load_gather(
            xv, [rows, jnp.full((L,), 1, jnp.int32)]).astype(jnp.int32)
        ib = plsc.load_gather(
            xv, [rows, jnp.full((L,), 2, jnp.int32)]).astype(jnp.int32)
        for c in range(16):
            cc = jnp.full((L,), c, jnp.int32)
            plsc.store_scatter(char_v, [rows, cc],
                               plsc.load_gather(wc_v, [ic, cc]))
        for c in range(8):
            cc = jnp.full((L,), c, jnp.int32)
            plsc.store_scatter(role_v, [rows, cc],
                               plsc.load_gather(wr_v, [ir, cc]))
        for c in range(6):
            cc = jnp.full((L,), c, jnp.int32)
            plsc.store_scatter(buff_v, [rows, cc],
                               plsc.load_gather(wb_v, [ib, cc]))

    pltpu.sync_copy(char_v, out_char.at[pl.ds(base, RPW)])
    pltpu.sync_copy(role_v, out_role.at[pl.ds(base, RPW)])
    pltpu.sync_copy(buff_v, out_buff.at[pl.ds(base, RPW)])


def _states_body(x_ref, o_ref):
    o_ref[...] = x_ref[:, 3:76]


_STATES_BLOCK = 2048


def _states(x):
    return pl.pallas_call(
        _states_body,
        grid=(B // _STATES_BLOCK,),
        in_specs=[pl.BlockSpec((_STATES_BLOCK, 76), lambda i: (i, 0))],
        out_specs=pl.BlockSpec((_STATES_BLOCK, 73), lambda i: (i, 0)),
        out_shape=jax.ShapeDtypeStruct((B, 73), jnp.float32),
    )(x)


def kernel(x, W_char, W_role, W_buff):
    my_char, my_role, my_buff = _sc_embed(x, W_char, W_role, W_buff)
    my_states = _states(x)
    return (my_char, my_role, my_buff, my_states)


# speedup vs baseline: 2.4439x; 1.0921x over previous
import functools

import jax
import jax.numpy as jnp
from jax import lax
from jax.experimental import pallas as pl
from jax.experimental.pallas import tpu as pltpu
from jax.experimental.pallas import tpu_sc as plsc

B = 16384
NC, NS, L = 2, 16, 16
NW = NC * NS
RPW = B // NW
NG = RPW // L

_mesh = plsc.VectorSubcoreMesh(
    core_axis_name="c", subcore_axis_name="s", num_cores=NC, num_subcores=NS
)


_ABLK = 2048


def _a_body(x_ref, states_ref, idx_ref):
    states_ref[...] = x_ref[:, 3:76]
    for k in range(3):
        idx_ref[k] = x_ref[:, k].astype(jnp.int32).reshape(_ABLK // 128, 128)


def _a_call(x):
    return pl.pallas_call(
        _a_body,
        grid=(B // _ABLK,),
        in_specs=[pl.BlockSpec((_ABLK, 76), lambda i: (i, 0))],
        out_specs=[
            pl.BlockSpec((_ABLK, 73), lambda i: (i, 0)),
            pl.BlockSpec((3, _ABLK // 128, 128), lambda i: (0, i, 0)),
        ],
        out_shape=[
            jax.ShapeDtypeStruct((B, 73), jnp.float32),
            jax.ShapeDtypeStruct((3, B // 128, 128), jnp.int32),
        ],
    )(x)


IB = RPW // 128


@functools.partial(
    pl.kernel,
    out_type=(
        jax.ShapeDtypeStruct((16, B), jnp.float32),
        jax.ShapeDtypeStruct((8, B), jnp.float32),
        jax.ShapeDtypeStruct((8, B), jnp.float32),
    ),
    mesh=_mesh,
    compiler_params=pltpu.CompilerParams(
        use_tc_tiling_on_sc=False, needs_layout_passes=False
    ),
    scratch_types=[
        pltpu.VMEM((IB, 128), jnp.int32),
        pltpu.VMEM((IB, 128), jnp.int32),
        pltpu.VMEM((IB, 128), jnp.int32),
        pltpu.VMEM((16, RPW), jnp.float32),
        pltpu.VMEM((8, RPW), jnp.float32),
        pltpu.VMEM((8, RPW), jnp.float32),
        pltpu.VMEM((16, 100), jnp.float32),
        pltpu.VMEM((8, 8), jnp.float32),
        pltpu.VMEM((6, 50), jnp.float32),
    ],
)
def _sc_embed(idx_hbm, wcT_hbm, wrT_hbm, wbT_hbm,
              out_charT, out_roleT, out_buffT,
              idxc_v, idxr_v, idxb_v, charT_v, roleT_v, buffT_v,
              wcT_v, wrT_v, wbT_v):
    wid = lax.axis_index("s") * NC + lax.axis_index("c")
    base = wid * RPW

    pltpu.sync_copy(idx_hbm.at[0, pl.ds(wid * IB, IB)], idxc_v)
    pltpu.sync_copy(idx_hbm.at[1, pl.ds(wid * IB, IB)], idxr_v)
    pltpu.sync_copy(idx_hbm.at[2, pl.ds(wid * IB, IB)], idxb_v)
    pltpu.sync_copy(wcT_hbm, wcT_v)
    pltpu.sync_copy(wrT_hbm, wrT_v)
    pltpu.sync_copy(wbT_hbm, wbT_v)

    for g in range(NG):
        sl = pl.ds((g % 8) * L, L)
        ic = idxc_v[g // 8, sl]
        ir = idxr_v[g // 8, sl]
        ib = idxb_v[g // 8, sl]
        out_sl = pl.ds(g * L, L)
        for c in range(16):
            cc = jnp.full((L,), c, jnp.int32)
            charT_v[c, out_sl] = plsc.load_gather(wcT_v, [cc, ic])
        for c in range(8):
            cc = jnp.full((L,), c, jnp.int32)
            roleT_v[c, out_sl] = plsc.load_gather(wrT_v, [cc, ir])
        for c in range(6):
            cc = jnp.full((L,), c, jnp.int32)
            buffT_v[c, out_sl] = plsc.load_gather(wbT_v, [cc, ib])

    pltpu.sync_copy(charT_v, out_charT.at[:, pl.ds(base, RPW)])
    pltpu.sync_copy(roleT_v, out_roleT.at[:, pl.ds(base, RPW)])
    pltpu.sync_copy(buffT_v, out_buffT.at[:, pl.ds(base, RPW)])


_CBLK = 2048


def _c_body(charT_ref, roleT_ref, buffT_ref, char_ref, role_ref, buff_ref):
    char_ref[...] = charT_ref[...].T
    role_ref[...] = roleT_ref[...].T
    buff_ref[...] = buffT_ref[0:6, :].T


def _c_call(charT, roleT, buffT):
    return pl.pallas_call(
        _c_body,
        grid=(B // _CBLK,),
        in_specs=[
            pl.BlockSpec((16, _CBLK), lambda i: (0, i)),
            pl.BlockSpec((8, _CBLK), lambda i: (0, i)),
            pl.BlockSpec((8, _CBLK), lambda i: (0, i)),
        ],
        out_specs=[
            pl.BlockSpec((_CBLK, 16), lambda i: (i, 0)),
            pl.BlockSpec((_CBLK, 8), lambda i: (i, 0)),
            pl.BlockSpec((_CBLK, 6), lambda i: (i, 0)),
        ],
        out_shape=[
            jax.ShapeDtypeStruct((B, 16), jnp.float32),
            jax.ShapeDtypeStruct((B, 8), jnp.float32),
            jax.ShapeDtypeStruct((B, 6), jnp.float32),
        ],
    )(charT, roleT, buffT)


def kernel(x, W_char, W_role, W_buff):
    my_states, idx3 = _a_call(x)
    charT, roleT, buffT = _sc_embed(idx3, W_char.T, W_role.T, W_buff.T)
    my_char, my_role, my_buff = _c_call(charT, roleT, buffT)
    return (my_char, my_role, my_buff, my_states)


# speedup vs baseline: 5.3168x; 2.1756x over previous
import functools

import jax
import jax.numpy as jnp
from jax import lax
from jax.experimental import pallas as pl
from jax.experimental.pallas import tpu as pltpu
from jax.experimental.pallas import tpu_sc as plsc

B = 16384
NC, NS, L = 2, 16, 16
NW = NC * NS
RPW = B // NW
NG = RPW // L

_mesh = plsc.VectorSubcoreMesh(
    core_axis_name="c", subcore_axis_name="s", num_cores=NC, num_subcores=NS
)


IB = RPW // 128


@functools.partial(
    pl.kernel,
    out_type=(
        jax.ShapeDtypeStruct((16, B), jnp.float32),
        jax.ShapeDtypeStruct((8, B), jnp.float32),
        jax.ShapeDtypeStruct((8, B), jnp.float32),
    ),
    mesh=_mesh,
    compiler_params=pltpu.CompilerParams(
        use_tc_tiling_on_sc=False, needs_layout_passes=False
    ),
    scratch_types=[
        pltpu.VMEM((IB, 128), jnp.int32),
        pltpu.VMEM((IB, 128), jnp.int32),
        pltpu.VMEM((IB, 128), jnp.int32),
        pltpu.VMEM((16, RPW), jnp.float32),
        pltpu.VMEM((8, RPW), jnp.float32),
        pltpu.VMEM((8, RPW), jnp.float32),
        pltpu.VMEM((16, 100), jnp.float32),
        pltpu.VMEM((8, 8), jnp.float32),
        pltpu.VMEM((6, 50), jnp.float32),
    ],
)
def _sc_embed(idx_hbm, wcT_hbm, wrT_hbm, wbT_hbm,
              out_charT, out_roleT, out_buffT,
              idxc_v, idxr_v, idxb_v, charT_v, roleT_v, buffT_v,
              wcT_v, wrT_v, wbT_v):
    wid = lax.axis_index("s") * NC + lax.axis_index("c")
    base = wid * RPW

    pltpu.sync_copy(idx_hbm.at[0, pl.ds(wid * IB, IB)], idxc_v)
    pltpu.sync_copy(idx_hbm.at[1, pl.ds(wid * IB, IB)], idxr_v)
    pltpu.sync_copy(idx_hbm.at[2, pl.ds(wid * IB, IB)], idxb_v)
    pltpu.sync_copy(wcT_hbm, wcT_v)
    pltpu.sync_copy(wrT_hbm, wrT_v)
    pltpu.sync_copy(wbT_hbm, wbT_v)

    for g in range(NG):
        sl = pl.ds((g % 8) * L, L)
        ic = idxc_v[g // 8, sl]
        ir = idxr_v[g // 8, sl]
        ib = idxb_v[g // 8, sl]
        out_sl = pl.ds(g * L, L)
        for c in range(16):
            cc = jnp.full((L,), c, jnp.int32)
            charT_v[c, out_sl] = plsc.load_gather(wcT_v, [cc, ic])
        for c in range(8):
            cc = jnp.full((L,), c, jnp.int32)
            roleT_v[c, out_sl] = plsc.load_gather(wrT_v, [cc, ir])
        for c in range(6):
            cc = jnp.full((L,), c, jnp.int32)
            buffT_v[c, out_sl] = plsc.load_gather(wbT_v, [cc, ib])

    pltpu.sync_copy(charT_v, out_charT.at[:, pl.ds(base, RPW)])
    pltpu.sync_copy(roleT_v, out_roleT.at[:, pl.ds(base, RPW)])
    pltpu.sync_copy(buffT_v, out_buffT.at[:, pl.ds(base, RPW)])


def kernel(x, W_char, W_role, W_buff):
    idx3 = x[:, 0:3].astype(jnp.int32).T.reshape(3, B // 128, 128)
    charT, roleT, buffT = _sc_embed(idx3, W_char.T, W_role.T, W_buff.T)
    my_char = charT.T
    my_role = roleT.T
    my_buff = buffT[0:6, :].T
    my_states = x[:, 3:76]
    return (my_char, my_role, my_buff, my_states)


# speedup vs baseline: 5.8664x; 1.1034x over previous
import functools

import jax
import jax.numpy as jnp
from jax import lax
from jax.experimental import pallas as pl
from jax.experimental.pallas import tpu as pltpu
from jax.experimental.pallas import tpu_sc as plsc

B = 16384
NC, NS, L = 2, 16, 16
NW = NC * NS
RPW = B // NW
IB = RPW // 128

_mesh = plsc.VectorSubcoreMesh(
    core_axis_name="c", subcore_axis_name="s", num_cores=NC, num_subcores=NS
)


@functools.partial(
    pl.kernel,
    out_type=(
        jax.ShapeDtypeStruct((16, B), jnp.float32),
        jax.ShapeDtypeStruct((8, B), jnp.float32),
        jax.ShapeDtypeStruct((8, B), jnp.float32),
    ),
    mesh=_mesh,
    compiler_params=pltpu.CompilerParams(
        use_tc_tiling_on_sc=False, needs_layout_passes=False
    ),
    scratch_types=[
        pltpu.VMEM((IB, 128), jnp.int32),
        pltpu.VMEM((IB, 128), jnp.int32),
        pltpu.VMEM((IB, 128), jnp.int32),
        pltpu.VMEM((32, 128), jnp.float32),
        pltpu.VMEM((16, RPW), jnp.float32),
        pltpu.VMEM((8, RPW), jnp.float32),
        pltpu.VMEM((8, RPW), jnp.float32),
        pltpu.SemaphoreType.DMA,
        pltpu.SemaphoreType.DMA,
    ],
)
def _sc_embed(idx_hbm, wt_hbm,
              out_charT, out_roleT, out_buffT,
              idxc_v, idxr_v, idxb_v, wt_v, charT_v, roleT_v, buffT_v,
              sem_in, sem_out):
    wid = lax.axis_index("s") * NC + lax.axis_index("c")
    base = wid * RPW

    cps = [
        pltpu.async_copy(idx_hbm.at[0, pl.ds(wid * IB, IB)], idxc_v, sem_in),
        pltpu.async_copy(idx_hbm.at[1, pl.ds(wid * IB, IB)], idxr_v, sem_in),
        pltpu.async_copy(idx_hbm.at[2, pl.ds(wid * IB, IB)], idxb_v, sem_in),
        pltpu.async_copy(wt_hbm, wt_v, sem_in),
    ]
    for cp in cps:
        cp.wait()

    @plsc.parallel_loop(0, IB)
    def _row_block(j):
        for k in range(8):
            sl = pl.ds(k * L, L)
            ic = idxc_v[j, sl]
            ir = idxr_v[j, sl]
            ib = idxb_v[j, sl]
            out_sl = pl.ds(j * 128 + k * L, L)
            for c in range(16):
                cc = jnp.full((L,), c, jnp.int32)
                charT_v[c, out_sl] = plsc.load_gather(wt_v, [cc, ic])
            for c in range(8):
                cc = jnp.full((L,), 16 + c, jnp.int32)
                roleT_v[c, out_sl] = plsc.load_gather(wt_v, [cc, ir])
            for c in range(6):
                cc = jnp.full((L,), 24 + c, jnp.int32)
                buffT_v[c, out_sl] = plsc.load_gather(wt_v, [cc, ib])

    outs = [
        pltpu.async_copy(charT_v, out_charT.at[:, pl.ds(base, RPW)], sem_out),
        pltpu.async_copy(roleT_v, out_roleT.at[:, pl.ds(base, RPW)], sem_out),
        pltpu.async_copy(buffT_v, out_buffT.at[:, pl.ds(base, RPW)], sem_out),
    ]
    for cp in outs:
        cp.wait()


def kernel(x, W_char, W_role, W_buff):
    idx3 = x[:, 0:3].astype(jnp.int32).T.reshape(3, B // 128, 128)
    wt = jnp.zeros((32, 128), jnp.float32)
    wt = wt.at[0:16, 0:100].set(W_char.T)
    wt = wt.at[16:24, 0:8].set(W_role.T)
    wt = wt.at[24:30, 0:50].set(W_buff.T)
    charT, roleT, buffT = _sc_embed(idx3, wt)
    my_char = charT.T
    my_role = roleT.T
    my_buff = buffT[0:6, :].T
    my_states = x[:, 3:76]
    return (my_char, my_role, my_buff, my_states)


# speedup vs baseline: 6.6957x; 1.1414x over previous
import functools

import jax
import jax.numpy as jnp
from jax import lax
from jax.experimental import pallas as pl
from jax.experimental.pallas import tpu as pltpu
from jax.experimental.pallas import tpu_sc as plsc

B = 16384
NC, NS, L = 2, 16, 16
NW = NC * NS
RPW = B // NW
IB = RPW // 128

_mesh = plsc.VectorSubcoreMesh(
    core_axis_name="c", subcore_axis_name="s", num_cores=NC, num_subcores=NS
)


@functools.partial(
    pl.kernel,
    out_type=(
        jax.ShapeDtypeStruct((16, B), jnp.float32),
        jax.ShapeDtypeStruct((8, B), jnp.float32),
        jax.ShapeDtypeStruct((8, B), jnp.float32),
    ),
    mesh=_mesh,
    compiler_params=pltpu.CompilerParams(
        use_tc_tiling_on_sc=False,
        needs_layout_passes=False,
        disable_bounds_checks=True,
        disable_semaphore_checks=True,
        skip_device_barrier=True,
    ),
    scratch_types=[
        pltpu.VMEM((IB, 128), jnp.int32),
        pltpu.VMEM((IB, 128), jnp.int32),
        pltpu.VMEM((IB, 128), jnp.int32),
        pltpu.VMEM((32, 128), jnp.float32),
        pltpu.VMEM((16, RPW), jnp.float32),
        pltpu.VMEM((8, RPW), jnp.float32),
        pltpu.VMEM((8, RPW), jnp.float32),
        pltpu.SemaphoreType.DMA,
        pltpu.SemaphoreType.DMA,
    ],
)
def _sc_embed(idx_hbm, wt_hbm,
              out_charT, out_roleT, out_buffT,
              idxc_v, idxr_v, idxb_v, wt_v, charT_v, roleT_v, buffT_v,
              sem_in, sem_out):
    wid = lax.axis_index("s") * NC + lax.axis_index("c")
    base = wid * RPW

    cps = [
        pltpu.async_copy(idx_hbm.at[0, pl.ds(wid * IB, IB)], idxc_v, sem_in),
        pltpu.async_copy(idx_hbm.at[1, pl.ds(wid * IB, IB)], idxr_v, sem_in),
        pltpu.async_copy(idx_hbm.at[2, pl.ds(wid * IB, IB)], idxb_v, sem_in),
        pltpu.async_copy(wt_hbm, wt_v, sem_in),
    ]
    for cp in cps:
        cp.wait()

    @plsc.parallel_loop(0, IB)
    def _row_block(j):
        @plsc.parallel_loop(0, 8)
        def _group(k):
            sl = pl.ds(k * L, L)
            ic = idxc_v[j, sl]
            ir = idxr_v[j, sl]
            ib = idxb_v[j, sl]
            out_sl = pl.ds(j * 128 + k * L, L)
            for c in range(16):
                cc = jnp.full((L,), c, jnp.int32)
                charT_v[c, out_sl] = plsc.load_gather(wt_v, [cc, ic])
            for c in range(8):
                cc = jnp.full((L,), 16 + c, jnp.int32)
                roleT_v[c, out_sl] = plsc.load_gather(wt_v, [cc, ir])
            for c in range(6):
                cc = jnp.full((L,), 24 + c, jnp.int32)
                buffT_v[c, out_sl] = plsc.load_gather(wt_v, [cc, ib])

    outs = [
        pltpu.async_copy(charT_v, out_charT.at[:, pl.ds(base, RPW)], sem_out),
        pltpu.async_copy(roleT_v, out_roleT.at[:, pl.ds(base, RPW)], sem_out),
        pltpu.async_copy(buffT_v, out_buffT.at[:, pl.ds(base, RPW)], sem_out),
    ]
    for cp in outs:
        cp.wait()


def kernel(x, W_char, W_role, W_buff):
    idx3 = x[:, 0:3].astype(jnp.int32).T.reshape(3, B // 128, 128)
    wt = jnp.concatenate([
        jnp.pad(W_char.T, ((0, 0), (0, 28))),
        jnp.pad(W_role.T, ((0, 0), (0, 120))),
        jnp.pad(W_buff.T, ((0, 2), (0, 78))),
    ], axis=0)
    charT, roleT, buffT = _sc_embed(idx3, wt)
    my_char = charT.T
    my_role = roleT.T
    my_buff = buffT[0:6, :].T
    my_states = x[:, 3:76]
    return (my_char, my_role, my_buff, my_states)
